# Initial kernel scaffold; baseline (speedup 1.0000x reference)
import jax
import jax.numpy as jnp
from jax.experimental import pallas as pl


def kernel(queries, keys, values, attn_mask):
    raise NotImplementedError("write your pallas kernel here")


# speedup vs baseline: 3.4468x; 3.4468x over previous
import math

import numpy as np
import jax
import jax.numpy as jnp
from jax import lax
from jax.experimental import pallas as pl
from jax.experimental.pallas import tpu as pltpu

_FACTOR = 5


def _period_sample_indices(L_K, sample_k):
    periods = [24.0, 12.0, 168.0, 8.0]
    usable = [p for p in periods if p <= L_K]
    out = []
    for p in usable:
        for i in range(math.ceil(L_K / p)):
            if L_K - (i + 1) * p >= 0:
                out.append(int(L_K - (i + 1) * p))
    return out[:sample_k]


def _make_body(L, D, HB, u_part, u, n_chunk):
    C = L // n_chunk
    scale = 1.0 / math.sqrt(D)

    def body(s_ref, q_hbm, k_hbm, v_hbm, o_hbm,
             qv, kv, vv, ov, sel_ref, sem_in, sem_out):
        b = pl.program_id(0)
        g = pl.program_id(1)

        copies = []
        for j in range(HB):
            h = g * HB + j
            copies.append(pltpu.make_async_copy(q_hbm.at[b, :, h, :], qv.at[j], sem_in))
            copies.append(pltpu.make_async_copy(k_hbm.at[b, :, h, :], kv.at[j], sem_in))
            copies.append(pltpu.make_async_copy(v_hbm.at[b, :, h, :], vv.at[j], sem_in))
        for cp in copies:
            cp.start()
        for cp in copies:
            cp.wait()

        S = s_ref[...]
        iota_l = lax.broadcasted_iota(jnp.int32, (HB, L), 1)

        m_rows = []
        for j in range(HB):
            Qj = qv[j]
            Kj = kv[j]
            Ksel = lax.dot_general(S, Kj, (((1,), (0,)), ((), ())),
                                   precision=lax.Precision.HIGHEST,
                                   preferred_element_type=jnp.float32)
            Ksq = Ksel * Ksel
            QKT = lax.dot_general(Ksq, Qj, (((1,), (1,)), ((), ())),
                                  preferred_element_type=jnp.float32)
            Mj = (jnp.max(QKT, axis=0, keepdims=True)
                  - jnp.sum(QKT, axis=0, keepdims=True) * (1.0 / L))
            m_rows.append(Mj)
        m0 = jnp.concatenate(m_rows, axis=0)

        def tk_body(i, m):
            mx = jnp.max(m, axis=1, keepdims=True)
            ismax = m >= mx
            fidx = jnp.min(jnp.where(ismax, iota_l, L), axis=1, keepdims=True)
            onehot = iota_l == fidx
            sel_ref[pl.ds(i, 1), :, :] = onehot.astype(jnp.float32)[None]
            return jnp.where(onehot, -jnp.inf, m)

        lax.fori_loop(0, u, tk_body, m0)

        tri = (lax.broadcasted_iota(jnp.int32, (C, C), 0)
               >= lax.broadcasted_iota(jnp.int32, (C, C), 1)).astype(jnp.float32)
        ones_u = jnp.ones((u, 1), jnp.float32)
        iota_col = lax.broadcasted_iota(jnp.int32, (L, 1), 0).astype(jnp.float32)
        iota_ul = lax.broadcasted_iota(jnp.int32, (u, L), 1)

        for j in range(HB):
            Qj = qv[j]
            Kj = kv[j]
            selj = sel_ref[:, j, :]
            qpos = lax.dot_general(selj, iota_col, (((1,), (0,)), ((), ())),
                                   precision=lax.Precision.HIGHEST,
                                   preferred_element_type=jnp.float32)
            valj = iota_ul <= qpos.astype(jnp.int32)

            Qred = lax.dot_general(selj, Qj, (((1,), (0,)), ((), ())),
                                   precision=lax.Precision.HIGHEST,
                                   preferred_element_type=jnp.float32)
            sc = lax.dot_general(Qred, Kj, (((1,), (1,)), ((), ())),
                                 preferred_element_type=jnp.float32) * scale
            sc = jnp.where(valj, sc, -jnp.inf)
            sc = sc - jnp.max(sc, axis=1, keepdims=True)
            e = jnp.exp(sc)
            attn = e / jnp.sum(e, axis=1, keepdims=True)
            upd = lax.dot_general(attn, vv[j], (((1,), (0,)), ((), ())),
                                  precision=lax.Precision.HIGHEST,
                                  preferred_element_type=jnp.float32)

            carry = jnp.zeros((1, D), jnp.float32)
            for c in range(n_chunk):
                vc = vv[j, c * C:(c + 1) * C, :]
                cs = lax.dot_general(tri, vc, (((1,), (0,)), ((), ())),
                                     precision=lax.Precision.HIGHEST,
                                     preferred_element_type=jnp.float32) + carry
                carry = cs[C - 1:C, :]
                selc = selj[:, c * C:(c + 1) * C]
                scat = lax.dot_general(selc, upd, (((0,), (0,)), ((), ())),
                                       precision=lax.Precision.HIGHEST,
                                       preferred_element_type=jnp.float32)
                mcol = lax.dot_general(selc, ones_u, (((0,), (0,)), ((), ())),
                                       precision=lax.Precision.HIGHEST,
                                       preferred_element_type=jnp.float32)
                ov[j, c * C:(c + 1) * C, :] = cs * (1.0 - mcol) + scat

        outs = []
        for j in range(HB):
            h = g * HB + j
            outs.append(pltpu.make_async_copy(ov.at[j], o_hbm.at[b, :, h, :], sem_out))
        for cp in outs:
            cp.start()
        for cp in outs:
            cp.wait()

    return body


def kernel(queries, keys, values, attn_mask):
    B, L, H, D = queries.shape
    L_K = keys.shape[1]
    u_part = min(_FACTOR * int(np.ceil(np.log(L_K))), L_K)
    u = min(_FACTOR * int(np.ceil(np.log(L))), L)
    idx = _period_sample_indices(L_K, u_part)
    S_np = np.zeros((u_part, L_K), np.float32)
    S_np[np.arange(u_part), np.asarray(idx)] = 1.0
    S = jnp.asarray(S_np)

    HB = 8 if H % 8 == 0 else H
    n_hb = H // HB
    n_chunk = 8

    out = pl.pallas_call(
        _make_body(L, D, HB, u_part, u, n_chunk),
        grid=(B, n_hb),
        in_specs=[
            pl.BlockSpec((u_part, L_K), lambda b, g: (0, 0)),
            pl.BlockSpec(memory_space=pl.ANY),
            pl.BlockSpec(memory_space=pl.ANY),
            pl.BlockSpec(memory_space=pl.ANY),
        ],
        out_specs=pl.BlockSpec(memory_space=pl.ANY),
        out_shape=jax.ShapeDtypeStruct((B, L, H, D), jnp.float32),
        scratch_shapes=[
            pltpu.VMEM((HB, L, D), jnp.float32),
            pltpu.VMEM((HB, L_K, D), jnp.float32),
            pltpu.VMEM((HB, L_K, D), jnp.float32),
            pltpu.VMEM((HB, L, D), jnp.float32),
            pltpu.VMEM((u, HB, L), jnp.float32),
            pltpu.SemaphoreType.DMA,
            pltpu.SemaphoreType.DMA,
        ],
        compiler_params=pltpu.CompilerParams(
            dimension_semantics=("arbitrary", "arbitrary")),
    )(S, queries, keys, values)
    return out
